# Initial kernel scaffold; baseline (speedup 1.0000x reference)
import jax
import jax.numpy as jnp
from jax.experimental import pallas as pl


def kernel(x, new_supernode_x, supernode_edge_index, supernode_idx, graph_batch, W1, b1, W2, b2):
    raise NotImplementedError("write your pallas kernel here")


# speedup vs baseline: 2.6681x; 2.6681x over previous
import functools

import jax
import jax.numpy as jnp
from jax import lax
from jax.experimental import pallas as pl
from jax.experimental.pallas import tpu as pltpu
from jax.experimental.pallas import tpu_sc as plsc

N = 100000
S = 10000
E = 320000
D = 128

NSUB = 16
CHUNK = 10240
NPASS = 5
RPW = CHUNK // NSUB
NDUMP = 16

SN_PAD = 10240
SN_PW = SN_PAD // NSUB
E_PW = E // NSUB
EBLK = 2000
NBLK = E_PW // EBLK

B = 96
KCAP_A = ((SN_PW + B - 1) // B) * B
KCAP_B = ((EBLK + B - 1) // B) * B
NPAD = 2 * NPASS * CHUNK

_SENTINEL = 2 ** 30


def _fill2d(buf, nrows, val_v, lane):
    shift = (buf.shape[1] - 1).bit_length()
    assert buf.shape[1] == 1 << shift

    def body(i, _):
        f = i * 16 + lane
        plsc.store_scatter(buf, [f >> shift, f & (buf.shape[1] - 1)], val_v)
        return 0

    lax.fori_loop(0, nrows * buf.shape[1] // 16, body, 0)


def _bounce(src_at, dst_at, buf, nrows):
    nb = buf.shape[0]
    for t in range(0, nrows - nrows % nb, nb):
        pltpu.sync_copy(src_at(t, nb), buf)
        pltpu.sync_copy(buf, dst_at(t, nb))
    rem = nrows % nb
    if rem:
        t = nrows - rem
        pltpu.sync_copy(src_at(t, rem), buf.at[pl.ds(0, rem)])
        pltpu.sync_copy(buf.at[pl.ds(0, rem)], dst_at(t, rem))


def _prefill(kf, relf, nvec, safe_v, dump_v):
    def body(i, _):
        kf[pl.ds(i * 16, 16)] = safe_v
        relf[pl.ds(i * 16, 16)] = dump_v
        return 0

    lax.fori_loop(0, nvec, body, 0)


def _compact(src_ids, dst_ids, kf, relf, nvec, lo, chunk):
    def body(i, cnt):
        dv = dst_ids(i)
        relv = dv - lo
        mask = (relv >= 0) & (relv < chunk)
        mi = jnp.where(mask, 1, 0).astype(jnp.int32)
        pc = plsc.cumsum(mi)
        offs = cnt + pc - 1
        sv = src_ids(i)
        plsc.store_scatter(relf, [offs], relv, mask=mask)
        plsc.store_scatter(kf, [offs], sv, mask=mask)
        return cnt + jnp.sum(mi)

    return lax.fori_loop(0, nvec, body, jnp.int32(0))


def _sweep_a(x, proj1, sidx_pad):
    mesh = plsc.VectorSubcoreMesh(core_axis_name="c", subcore_axis_name="s")

    @functools.partial(
        pl.kernel,
        out_type=jax.ShapeDtypeStruct((N, D), jnp.float32),
        mesh=mesh,
        compiler_params=pltpu.CompilerParams(needs_layout_passes=False),
        scratch_types=[
            pltpu.VMEM_SHARED((CHUNK + NDUMP, D), jnp.float32),
            pltpu.VMEM((SN_PW,), jnp.int32),
            pltpu.VMEM((KCAP_A,), jnp.int32),
            pltpu.VMEM((KCAP_A,), jnp.int32),
            pltpu.VMEM((B, D), jnp.float32),
        ],
    )
    def k(x_hbm, p1_hbm, sidx_hbm, out_hbm, sh_chunk, sids, kf, relf, row_buf):
        c = lax.axis_index("c")
        s = lax.axis_index("s")
        lane = lax.iota(jnp.int32, 16)
        pltpu.sync_copy(sidx_hbm.at[pl.ds(s * SN_PW, SN_PW)], sids)
        safe_v = s * 16 + lane
        dump_v = jnp.full((16,), CHUNK, jnp.int32) + s

        def pass_body(p, _):
            lo = (c * NPASS + p) * CHUNK
            g0 = lo + s * RPW
            cg = jnp.minimum(g0, N - RPW)
            cl = cg - lo
            _bounce(lambda t, n: x_hbm.at[pl.ds(cg + t, n)],
                    lambda t, n: sh_chunk.at[pl.ds(cl + t, n)], row_buf, RPW)
            _prefill(kf, relf, KCAP_A // 16, safe_v, dump_v)
            plsc.subcore_barrier()

            m = _compact(
                lambda i: s * SN_PW + i * 16 + lane,
                lambda i: sids[pl.ds(i * 16, 16)],
                kf, relf, SN_PW // 16, lo, CHUNK)

            def bat(b, _):
                pltpu.sync_copy(p1_hbm.at[kf.at[pl.ds(b * B, B)]], row_buf)
                pltpu.sync_copy(row_buf,
                                sh_chunk.at[relf.at[pl.ds(b * B, B)]], add=True)
                return 0

            lax.fori_loop(0, (m + B - 1) // B, bat, 0)
            plsc.subcore_barrier()
            _bounce(lambda t, n: sh_chunk.at[pl.ds(cl + t, n)],
                    lambda t, n: out_hbm.at[pl.ds(cg + t, n)], row_buf, RPW)
            plsc.subcore_barrier()
            return 0

        lax.fori_loop(0, NPASS, pass_body, 0)

    return k(x, proj1, sidx_pad)


def _sweep_b(x_mid, e0, e1):
    mesh = plsc.VectorSubcoreMesh(core_axis_name="c", subcore_axis_name="s")

    @functools.partial(
        pl.kernel,
        out_type=(
            jax.ShapeDtypeStruct((N, D), jnp.float32),
            jax.ShapeDtypeStruct((NSUB * NPAD,), jnp.float32),
        ),
        mesh=mesh,
        compiler_params=pltpu.CompilerParams(needs_layout_passes=False),
        scratch_types=[
            pltpu.VMEM_SHARED((CHUNK + NDUMP, D), jnp.float32),
            pltpu.VMEM((EBLK,), jnp.int32),
            pltpu.VMEM((EBLK,), jnp.int32),
            pltpu.VMEM((KCAP_B,), jnp.int32),
            pltpu.VMEM((KCAP_B,), jnp.int32),
            pltpu.VMEM((B, D), jnp.float32),
            pltpu.VMEM((CHUNK,), jnp.float32),
        ],
    )
    def k(xm_hbm, e0_hbm, e1_hbm, agg_hbm, deg_hbm,
          sh_chunk, e0s, e1s, kf, relf, row_buf, degloc):
        c = lax.axis_index("c")
        s = lax.axis_index("s")
        lane = lax.iota(jnp.int32, 16)
        zerof = jnp.zeros((16,), jnp.float32)
        safe_v = s * 16 + lane
        dump_v = jnp.full((16,), CHUNK, jnp.int32) + s

        def pass_body(p, _):
            lo = (c * NPASS + p) * CHUNK
            g0 = lo + s * RPW
            cg = jnp.minimum(g0, N - RPW)
            cl = cg - lo
            _fill2d(row_buf, B, zerof, lane)
            for t in range(RPW // B):
                pltpu.sync_copy(row_buf, sh_chunk.at[pl.ds(s * RPW + t * B, B)])
            rem = RPW % B
            if rem:
                pltpu.sync_copy(row_buf.at[pl.ds(0, rem)],
                                sh_chunk.at[pl.ds(s * RPW + (RPW // B) * B, rem)])

            def dz(i, _):
                degloc[pl.ds(i * 16, 16)] = zerof
                return 0

            lax.fori_loop(0, CHUNK // 16, dz, 0)
            plsc.subcore_barrier()

            def blk_body(blk, _):
                base = s * E_PW + blk * EBLK
                pltpu.sync_copy(e0_hbm.at[pl.ds(base, EBLK)], e0s)
                pltpu.sync_copy(e1_hbm.at[pl.ds(base, EBLK)], e1s)
                _prefill(kf, relf, KCAP_B // 16, safe_v, dump_v)

                def comp(i, cnt):
                    dv = e0s[pl.ds(i * 16, 16)]
                    relv = dv - lo
                    mask = (relv >= 0) & (relv < CHUNK)
                    mi = jnp.where(mask, 1, 0).astype(jnp.int32)
                    pc = plsc.cumsum(mi)
                    offs = cnt + pc - 1
                    sv = e1s[pl.ds(i * 16, 16)]
                    plsc.store_scatter(relf, [offs], relv, mask=mask)
                    plsc.store_scatter(kf, [offs], sv, mask=mask)
                    dcnt, lastm = plsc.scan_count(relv, mask=mask)
                    plsc.addupdate_scatter(
                        degloc, [relv], dcnt.astype(jnp.float32), mask=lastm)
                    return cnt + jnp.sum(mi)

                m = lax.fori_loop(0, EBLK // 16, comp, jnp.int32(0))

                def bat(b, _):
                    idx_b = relf.at[pl.ds(b * B, B)]
                    pltpu.sync_copy(xm_hbm.at[kf.at[pl.ds(b * B, B)]], row_buf)
                    pltpu.sync_copy(row_buf, sh_chunk.at[idx_b], add=True)
                    return 0

                lax.fori_loop(0, (m + B - 1) // B, bat, 0)
                return 0

            lax.fori_loop(0, NBLK, blk_body, 0)
            plsc.subcore_barrier()
            _bounce(lambda t, n: sh_chunk.at[pl.ds(cl + t, n)],
                    lambda t, n: agg_hbm.at[pl.ds(cg + t, n)], row_buf, RPW)
            pltpu.sync_copy(degloc, deg_hbm.at[pl.ds(s * NPAD + lo, CHUNK)])
            plsc.subcore_barrier()
            return 0

        lax.fori_loop(0, NPASS, pass_body, 0)

    return k(x_mid, e0, e1)


def _proj_tc(ns_x, W1, b1):
    def body(ns_ref, w_ref, b_ref, o_ref):
        o_ref[...] = lax.dot_general(
            ns_ref[...], w_ref[...], (((1,), (1,)), ((), ())),
            preferred_element_type=jnp.float32) + b_ref[...]

    blk = 1000
    return pl.pallas_call(
        body,
        grid=(S // blk,),
        in_specs=[
            pl.BlockSpec((blk, D), lambda i: (i, 0)),
            pl.BlockSpec((D, D), lambda i: (0, 0)),
            pl.BlockSpec((1, D), lambda i: (0, 0)),
        ],
        out_specs=pl.BlockSpec((blk, D), lambda i: (i, 0)),
        out_shape=jax.ShapeDtypeStruct((S, D), jnp.float32),
    )(ns_x, W1, b1.reshape(1, D))


def _final_tc(x_mid, agg, deg16, W2, b2):
    def body(xm_ref, agg_ref, deg_ref, w_ref, b_ref, o_ref):
        acc = lax.dot_general(
            agg_ref[...], w_ref[...], (((1,), (1,)), ((), ())),
            preferred_element_type=jnp.float32)
        ones16 = jnp.ones((NSUB, 1), jnp.float32)
        degcol = lax.dot_general(
            deg_ref[...], ones16, (((0,), (0,)), ((), ())),
            preferred_element_type=jnp.float32)
        o_ref[...] = xm_ref[...] + acc + degcol * b_ref[...]

    blk = 2048
    return pl.pallas_call(
        body,
        grid=((N + blk - 1) // blk,),
        in_specs=[
            pl.BlockSpec((blk, D), lambda i: (i, 0)),
            pl.BlockSpec((blk, D), lambda i: (i, 0)),
            pl.BlockSpec((NSUB, blk), lambda i: (0, i)),
            pl.BlockSpec((D, D), lambda i: (0, 0)),
            pl.BlockSpec((1, D), lambda i: (0, 0)),
        ],
        out_specs=pl.BlockSpec((blk, D), lambda i: (i, 0)),
        out_shape=jax.ShapeDtypeStruct((N, D), jnp.float32),
    )(x_mid, agg, deg16, W2, b2.reshape(1, D))


def _sweep_a_carcass(x):
    mesh = plsc.VectorSubcoreMesh(core_axis_name="c", subcore_axis_name="s")

    @functools.partial(
        pl.kernel,
        out_type=jax.ShapeDtypeStruct((N, D), jnp.float32),
        mesh=mesh,
        compiler_params=pltpu.CompilerParams(needs_layout_passes=False),
        scratch_types=[
            pltpu.VMEM_SHARED((CHUNK + NDUMP, D), jnp.float32),
            pltpu.VMEM((B, D), jnp.float32),
        ],
    )
    def k(x_hbm, out_hbm, sh_chunk, row_buf):
        c = lax.axis_index("c")
        s = lax.axis_index("s")

        def pass_body(p, _):
            lo = (c * NPASS + p) * CHUNK
            g0 = lo + s * RPW
            cg = jnp.minimum(g0, N - RPW)
            cl = cg - lo
            _bounce(lambda t, n: x_hbm.at[pl.ds(cg + t, n)],
                    lambda t, n: sh_chunk.at[pl.ds(cl + t, n)], row_buf, RPW)
            plsc.subcore_barrier()
            _bounce(lambda t, n: sh_chunk.at[pl.ds(cl + t, n)],
                    lambda t, n: out_hbm.at[pl.ds(cg + t, n)], row_buf, RPW)
            plsc.subcore_barrier()
            return 0

        lax.fori_loop(0, NPASS, pass_body, 0)

    return k(x)


def kernel(x, new_supernode_x, supernode_edge_index, supernode_idx,
           graph_batch, W1, b1, W2, b2):
    del graph_batch
    sidx = supernode_idx.astype(jnp.int32)
    e0 = supernode_edge_index[0].astype(jnp.int32)
    e1 = supernode_edge_index[1].astype(jnp.int32)

    sidx_pad = jnp.concatenate(
        [sidx, jnp.full((SN_PAD - S,), _SENTINEL, jnp.int32)])
    proj1 = _proj_tc(new_supernode_x, W1, b1)
    x_mid = _sweep_a(x, proj1, sidx_pad)
    agg, deg_flat = _sweep_b(x_mid, e0, e1)
    deg16 = deg_flat.reshape(NSUB, NPAD)
    return _final_tc(x_mid, agg, deg16, W2, b2)


# speedup vs baseline: 2.8534x; 1.0695x over previous
import functools

import jax
import jax.numpy as jnp
from jax import lax
from jax.experimental import pallas as pl
from jax.experimental.pallas import tpu as pltpu
from jax.experimental.pallas import tpu_sc as plsc

N = 100000
S = 10000
E = 320000
D = 128

NSUB = 16
CHUNK = 10240
NPASS = 5
RPW = CHUNK // NSUB
NDUMP = 16

SN_PAD = 10240
SN_PW = SN_PAD // NSUB
E_PW = E // NSUB
EBLK = 2000
NBLK = E_PW // EBLK

B = 96
KCAP_A = ((SN_PW + B - 1) // B) * B
KCAP_B = ((EBLK + B - 1) // B) * B
NPAD = 2 * NPASS * CHUNK

_SENTINEL = 2 ** 30


def _fill2d(buf, nrows, val_v, lane):
    shift = (buf.shape[1] - 1).bit_length()
    assert buf.shape[1] == 1 << shift

    def body(i, _):
        f = i * 16 + lane
        plsc.store_scatter(buf, [f >> shift, f & (buf.shape[1] - 1)], val_v)
        return 0

    lax.fori_loop(0, nrows * buf.shape[1] // 16, body, 0)


def _bounce(src_at, dst_at, buf, nrows):
    nb = buf.shape[0]
    for t in range(0, nrows - nrows % nb, nb):
        pltpu.sync_copy(src_at(t, nb), buf)
        pltpu.sync_copy(buf, dst_at(t, nb))
    rem = nrows % nb
    if rem:
        t = nrows - rem
        pltpu.sync_copy(src_at(t, rem), buf.at[pl.ds(0, rem)])
        pltpu.sync_copy(buf.at[pl.ds(0, rem)], dst_at(t, rem))


def _prefill(kf, relf, nvec, safe_v, dump_v):
    def body(i, _):
        kf[pl.ds(i * 16, 16)] = safe_v
        relf[pl.ds(i * 16, 16)] = dump_v
        return 0

    lax.fori_loop(0, nvec, body, 0)


def _compact(src_ids, dst_ids, kf, relf, nvec, lo, chunk):
    def body(i, cnt):
        dv = dst_ids(i)
        relv = dv - lo
        mask = (relv >= 0) & (relv < chunk)
        mi = jnp.where(mask, 1, 0).astype(jnp.int32)
        pc = plsc.cumsum(mi)
        offs = cnt + pc - 1
        sv = src_ids(i)
        plsc.store_scatter(relf, [offs], relv, mask=mask)
        plsc.store_scatter(kf, [offs], sv, mask=mask)
        return cnt + jnp.sum(mi)

    return lax.fori_loop(0, nvec, body, jnp.int32(0))


def _sweep_a(x, proj1, sidx_pad):
    mesh = plsc.VectorSubcoreMesh(core_axis_name="c", subcore_axis_name="s")

    @functools.partial(
        pl.kernel,
        out_type=jax.ShapeDtypeStruct((N, D), jnp.float32),
        mesh=mesh,
        compiler_params=pltpu.CompilerParams(needs_layout_passes=False),
        scratch_types=[
            pltpu.VMEM_SHARED((CHUNK + NDUMP, D), jnp.float32),
            pltpu.VMEM((SN_PW,), jnp.int32),
            pltpu.VMEM((KCAP_A,), jnp.int32),
            pltpu.VMEM((KCAP_A,), jnp.int32),
            pltpu.VMEM((B, D), jnp.float32),
        ],
    )
    def k(x_hbm, p1_hbm, sidx_hbm, out_hbm, sh_chunk, sids, kf, relf, row_buf):
        c = lax.axis_index("c")
        s = lax.axis_index("s")
        lane = lax.iota(jnp.int32, 16)
        pltpu.sync_copy(sidx_hbm.at[pl.ds(s * SN_PW, SN_PW)], sids)
        safe_v = s * 16 + lane
        dump_v = jnp.full((16,), CHUNK, jnp.int32) + s

        def pass_body(p, _):
            lo = (c * NPASS + p) * CHUNK
            g0 = lo + s * RPW
            cg = jnp.minimum(g0, N - RPW)
            cl = cg - lo
            _bounce(lambda t, n: x_hbm.at[pl.ds(cg + t, n)],
                    lambda t, n: sh_chunk.at[pl.ds(cl + t, n)], row_buf, RPW)
            _prefill(kf, relf, KCAP_A // 16, safe_v, dump_v)
            plsc.subcore_barrier()

            m = _compact(
                lambda i: s * SN_PW + i * 16 + lane,
                lambda i: sids[pl.ds(i * 16, 16)],
                kf, relf, SN_PW // 16, lo, CHUNK)

            def bat(b, _):
                pltpu.sync_copy(p1_hbm.at[kf.at[pl.ds(b * B, B)]], row_buf)
                pltpu.sync_copy(row_buf,
                                sh_chunk.at[relf.at[pl.ds(b * B, B)]], add=True)
                return 0

            lax.fori_loop(0, (m + B - 1) // B, bat, 0)
            plsc.subcore_barrier()
            _bounce(lambda t, n: sh_chunk.at[pl.ds(cl + t, n)],
                    lambda t, n: out_hbm.at[pl.ds(cg + t, n)], row_buf, RPW)
            plsc.subcore_barrier()
            return 0

        lax.fori_loop(0, NPASS, pass_body, 0)

    return k(x, proj1, sidx_pad)


def _sweep_b(x_mid, e0, e1):
    mesh = plsc.VectorSubcoreMesh(core_axis_name="c", subcore_axis_name="s")

    @functools.partial(
        pl.kernel,
        out_type=(
            jax.ShapeDtypeStruct((N, D), jnp.float32),
            jax.ShapeDtypeStruct((NSUB * NPAD,), jnp.float32),
        ),
        mesh=mesh,
        compiler_params=pltpu.CompilerParams(needs_layout_passes=False),
        scratch_types=[
            pltpu.VMEM_SHARED((CHUNK + NDUMP, D), jnp.float32),
            pltpu.VMEM((EBLK,), jnp.int32),
            pltpu.VMEM((EBLK,), jnp.int32),
            pltpu.VMEM((KCAP_B,), jnp.int32),
            pltpu.VMEM((KCAP_B,), jnp.int32),
            pltpu.VMEM((B, D), jnp.float32),
            pltpu.VMEM((B, D), jnp.float32),
            pltpu.VMEM((CHUNK,), jnp.float32),
            pltpu.SemaphoreType.DMA,
            pltpu.SemaphoreType.DMA,
        ],
    )
    def k(xm_hbm, e0_hbm, e1_hbm, agg_hbm, deg_hbm,
          sh_chunk, e0s, e1s, kf, relf, row_buf, row_buf2, degloc,
          semA, semB):
        c = lax.axis_index("c")
        s = lax.axis_index("s")
        lane = lax.iota(jnp.int32, 16)
        zerof = jnp.zeros((16,), jnp.float32)
        safe_v = s * 16 + lane
        dump_v = jnp.full((16,), CHUNK, jnp.int32) + s

        def pass_body(p, _):
            lo = (c * NPASS + p) * CHUNK
            g0 = lo + s * RPW
            cg = jnp.minimum(g0, N - RPW)
            cl = cg - lo
            _fill2d(row_buf, B, zerof, lane)
            for t in range(RPW // B):
                pltpu.sync_copy(row_buf, sh_chunk.at[pl.ds(s * RPW + t * B, B)])
            rem = RPW % B
            if rem:
                pltpu.sync_copy(row_buf.at[pl.ds(0, rem)],
                                sh_chunk.at[pl.ds(s * RPW + (RPW // B) * B, rem)])

            def dz(i, _):
                degloc[pl.ds(i * 16, 16)] = zerof
                return 0

            lax.fori_loop(0, CHUNK // 16, dz, 0)
            plsc.subcore_barrier()

            def blk_body(blk, _):
                base = s * E_PW + blk * EBLK
                pltpu.sync_copy(e0_hbm.at[pl.ds(base, EBLK)], e0s)
                pltpu.sync_copy(e1_hbm.at[pl.ds(base, EBLK)], e1s)
                _prefill(kf, relf, KCAP_B // 16, safe_v, dump_v)

                def comp(i, cnt):
                    dv = e0s[pl.ds(i * 16, 16)]
                    relv = dv - lo
                    mask = (relv >= 0) & (relv < CHUNK)
                    mi = jnp.where(mask, 1, 0).astype(jnp.int32)
                    pc = plsc.cumsum(mi)
                    offs = cnt + pc - 1
                    sv = e1s[pl.ds(i * 16, 16)]
                    plsc.store_scatter(relf, [offs], relv, mask=mask)
                    plsc.store_scatter(kf, [offs], sv, mask=mask)
                    dcnt, lastm = plsc.scan_count(relv, mask=mask)
                    plsc.addupdate_scatter(
                        degloc, [relv], dcnt.astype(jnp.float32), mask=lastm)
                    return cnt + jnp.sum(mi)

                m = lax.fori_loop(0, EBLK // 16, comp, jnp.int32(0))

                nb = (m + B - 1) // B

                def gather_src(b):
                    return xm_hbm.at[kf.at[pl.ds(b * B, B)]]

                def drain(b, buf, sem, nxt_buf, nxt_sem):
                    pltpu.make_async_copy(gather_src(b), buf, sem).wait()

                    @pl.when(b + 1 < nb)
                    def _():
                        pltpu.async_copy(gather_src(b + 1), nxt_buf, nxt_sem)

                    pltpu.sync_copy(buf, sh_chunk.at[relf.at[pl.ds(b * B, B)]],
                                    add=True)

                @pl.when(nb > 0)
                def _():
                    pltpu.async_copy(gather_src(0), row_buf, semA)

                def bat(b, _):
                    @pl.when(b % 2 == 0)
                    def _():
                        drain(b, row_buf, semA, row_buf2, semB)

                    @pl.when(b % 2 == 1)
                    def _():
                        drain(b, row_buf2, semB, row_buf, semA)

                    return 0

                lax.fori_loop(0, nb, bat, 0)
                return 0

            lax.fori_loop(0, NBLK, blk_body, 0)
            plsc.subcore_barrier()
            _bounce(lambda t, n: sh_chunk.at[pl.ds(cl + t, n)],
                    lambda t, n: agg_hbm.at[pl.ds(cg + t, n)], row_buf, RPW)
            pltpu.sync_copy(degloc, deg_hbm.at[pl.ds(s * NPAD + lo, CHUNK)])
            plsc.subcore_barrier()
            return 0

        lax.fori_loop(0, NPASS, pass_body, 0)

    return k(x_mid, e0, e1)


def _proj_tc(ns_x, W1, b1):
    def body(ns_ref, w_ref, b_ref, o_ref):
        o_ref[...] = lax.dot_general(
            ns_ref[...], w_ref[...], (((1,), (1,)), ((), ())),
            preferred_element_type=jnp.float32) + b_ref[...]

    blk = 1000
    return pl.pallas_call(
        body,
        grid=(S // blk,),
        in_specs=[
            pl.BlockSpec((blk, D), lambda i: (i, 0)),
            pl.BlockSpec((D, D), lambda i: (0, 0)),
            pl.BlockSpec((1, D), lambda i: (0, 0)),
        ],
        out_specs=pl.BlockSpec((blk, D), lambda i: (i, 0)),
        out_shape=jax.ShapeDtypeStruct((S, D), jnp.float32),
    )(ns_x, W1, b1.reshape(1, D))


def _final_tc(x_mid, agg, deg16, W2, b2):
    def body(xm_ref, agg_ref, deg_ref, w_ref, b_ref, o_ref):
        acc = lax.dot_general(
            agg_ref[...], w_ref[...], (((1,), (1,)), ((), ())),
            preferred_element_type=jnp.float32)
        ones16 = jnp.ones((NSUB, 1), jnp.float32)
        degcol = lax.dot_general(
            deg_ref[...], ones16, (((0,), (0,)), ((), ())),
            preferred_element_type=jnp.float32)
        o_ref[...] = xm_ref[...] + acc + degcol * b_ref[...]

    blk = 2048
    return pl.pallas_call(
        body,
        grid=((N + blk - 1) // blk,),
        in_specs=[
            pl.BlockSpec((blk, D), lambda i: (i, 0)),
            pl.BlockSpec((blk, D), lambda i: (i, 0)),
            pl.BlockSpec((NSUB, blk), lambda i: (0, i)),
            pl.BlockSpec((D, D), lambda i: (0, 0)),
            pl.BlockSpec((1, D), lambda i: (0, 0)),
        ],
        out_specs=pl.BlockSpec((blk, D), lambda i: (i, 0)),
        out_shape=jax.ShapeDtypeStruct((N, D), jnp.float32),
    )(x_mid, agg, deg16, W2, b2.reshape(1, D))


def _sweep_a_carcass(x):
    mesh = plsc.VectorSubcoreMesh(core_axis_name="c", subcore_axis_name="s")

    @functools.partial(
        pl.kernel,
        out_type=jax.ShapeDtypeStruct((N, D), jnp.float32),
        mesh=mesh,
        compiler_params=pltpu.CompilerParams(needs_layout_passes=False),
        scratch_types=[
            pltpu.VMEM_SHARED((CHUNK + NDUMP, D), jnp.float32),
            pltpu.VMEM((B, D), jnp.float32),
        ],
    )
    def k(x_hbm, out_hbm, sh_chunk, row_buf):
        c = lax.axis_index("c")
        s = lax.axis_index("s")

        def pass_body(p, _):
            lo = (c * NPASS + p) * CHUNK
            g0 = lo + s * RPW
            cg = jnp.minimum(g0, N - RPW)
            cl = cg - lo
            _bounce(lambda t, n: x_hbm.at[pl.ds(cg + t, n)],
                    lambda t, n: sh_chunk.at[pl.ds(cl + t, n)], row_buf, RPW)
            plsc.subcore_barrier()
            _bounce(lambda t, n: sh_chunk.at[pl.ds(cl + t, n)],
                    lambda t, n: out_hbm.at[pl.ds(cg + t, n)], row_buf, RPW)
            plsc.subcore_barrier()
            return 0

        lax.fori_loop(0, NPASS, pass_body, 0)

    return k(x)


def kernel(x, new_supernode_x, supernode_edge_index, supernode_idx,
           graph_batch, W1, b1, W2, b2):
    del graph_batch
    sidx = supernode_idx.astype(jnp.int32)
    e0 = supernode_edge_index[0].astype(jnp.int32)
    e1 = supernode_edge_index[1].astype(jnp.int32)

    sidx_pad = jnp.concatenate(
        [sidx, jnp.full((SN_PAD - S,), _SENTINEL, jnp.int32)])
    proj1 = _proj_tc(new_supernode_x, W1, b1)
    x_mid = _sweep_a(x, proj1, sidx_pad)
    agg, deg_flat = _sweep_b(x_mid, e0, e1)
    deg16 = deg_flat.reshape(NSUB, NPAD)
    return _final_tc(x_mid, agg, deg16, W2, b2)


# speedup vs baseline: 2.9218x; 1.0240x over previous
import functools

import jax
import jax.numpy as jnp
from jax import lax
from jax.experimental import pallas as pl
from jax.experimental.pallas import tpu as pltpu
from jax.experimental.pallas import tpu_sc as plsc

N = 100000
S = 10000
E = 320000
D = 128

NSUB = 16
CHUNK = 10240
NPASS = 5
RPW = CHUNK // NSUB
NDUMP = 16

SN_PAD = 10240
SN_PW = SN_PAD // NSUB
E_PW = E // NSUB
EBLK = 2000
NBLK = E_PW // EBLK

B = 96
KCAP_A = ((SN_PW + B - 1) // B) * B
KCAP_B = ((EBLK + B - 1) // B) * B
NPAD = 2 * NPASS * CHUNK

_SENTINEL = 2 ** 30


def _fill2d(buf, nrows, val_v, lane):
    shift = (buf.shape[1] - 1).bit_length()
    assert buf.shape[1] == 1 << shift

    def body(i, _):
        f = i * 16 + lane
        plsc.store_scatter(buf, [f >> shift, f & (buf.shape[1] - 1)], val_v)
        return 0

    lax.fori_loop(0, nrows * buf.shape[1] // 16, body, 0)


def _bounce(src_at, dst_at, buf, nrows):
    nb = buf.shape[0]
    for t in range(0, nrows - nrows % nb, nb):
        pltpu.sync_copy(src_at(t, nb), buf)
        pltpu.sync_copy(buf, dst_at(t, nb))
    rem = nrows % nb
    if rem:
        t = nrows - rem
        pltpu.sync_copy(src_at(t, rem), buf.at[pl.ds(0, rem)])
        pltpu.sync_copy(buf.at[pl.ds(0, rem)], dst_at(t, rem))


def _prefill(kf, relf, nvec, safe_v, dump_v):
    def body(i, _):
        kf[pl.ds(i * 16, 16)] = safe_v
        relf[pl.ds(i * 16, 16)] = dump_v
        return 0

    lax.fori_loop(0, nvec, body, 0)


def _compact(src_ids, dst_ids, kf, relf, nvec, lo, chunk):
    def body(i, cnt):
        dv = dst_ids(i)
        relv = dv - lo
        mask = (relv >= 0) & (relv < chunk)
        mi = jnp.where(mask, 1, 0).astype(jnp.int32)
        pc = plsc.cumsum(mi)
        offs = cnt + pc - 1
        sv = src_ids(i)
        plsc.store_scatter(relf, [offs], relv, mask=mask)
        plsc.store_scatter(kf, [offs], sv, mask=mask)
        return cnt + jnp.sum(mi)

    return lax.fori_loop(0, nvec, body, jnp.int32(0))


def _sweep_a(x, proj1, sidx_pad):
    mesh = plsc.VectorSubcoreMesh(core_axis_name="c", subcore_axis_name="s")

    @functools.partial(
        pl.kernel,
        out_type=jax.ShapeDtypeStruct((N, D), jnp.float32),
        mesh=mesh,
        compiler_params=pltpu.CompilerParams(needs_layout_passes=False),
        scratch_types=[
            pltpu.VMEM_SHARED((CHUNK + NDUMP, D), jnp.float32),
            pltpu.VMEM((SN_PW,), jnp.int32),
            pltpu.VMEM((KCAP_A,), jnp.int32),
            pltpu.VMEM((KCAP_A,), jnp.int32),
            pltpu.VMEM((256, D), jnp.float32),
        ],
    )
    def k(x_hbm, p1_hbm, sidx_hbm, out_hbm, sh_chunk, sids, kf, relf, row_buf):
        c = lax.axis_index("c")
        s = lax.axis_index("s")
        lane = lax.iota(jnp.int32, 16)
        pltpu.sync_copy(sidx_hbm.at[pl.ds(s * SN_PW, SN_PW)], sids)
        safe_v = s * 16 + lane
        dump_v = jnp.full((16,), CHUNK, jnp.int32) + s

        def pass_body(p, _):
            lo = (c * NPASS + p) * CHUNK
            g0 = lo + s * RPW
            cg = jnp.minimum(g0, N - RPW)
            cl = cg - lo
            _bounce(lambda t, n: x_hbm.at[pl.ds(cg + t, n)],
                    lambda t, n: sh_chunk.at[pl.ds(cl + t, n)], row_buf, RPW)
            _prefill(kf, relf, KCAP_A // 16, safe_v, dump_v)
            plsc.subcore_barrier()

            m = _compact(
                lambda i: s * SN_PW + i * 16 + lane,
                lambda i: sids[pl.ds(i * 16, 16)],
                kf, relf, SN_PW // 16, lo, CHUNK)

            def bat(b, _):
                rb = row_buf.at[pl.ds(0, B)]
                pltpu.sync_copy(p1_hbm.at[kf.at[pl.ds(b * B, B)]], rb)
                pltpu.sync_copy(rb,
                                sh_chunk.at[relf.at[pl.ds(b * B, B)]], add=True)
                return 0

            lax.fori_loop(0, (m + B - 1) // B, bat, 0)
            plsc.subcore_barrier()
            _bounce(lambda t, n: sh_chunk.at[pl.ds(cl + t, n)],
                    lambda t, n: out_hbm.at[pl.ds(cg + t, n)], row_buf, RPW)
            plsc.subcore_barrier()
            return 0

        lax.fori_loop(0, NPASS, pass_body, 0)

    return k(x, proj1, sidx_pad)


def _sweep_b(x_mid, e0, e1):
    mesh = plsc.VectorSubcoreMesh(core_axis_name="c", subcore_axis_name="s")

    @functools.partial(
        pl.kernel,
        out_type=(
            jax.ShapeDtypeStruct((N, D), jnp.float32),
            jax.ShapeDtypeStruct((NSUB * NPAD,), jnp.float32),
        ),
        mesh=mesh,
        compiler_params=pltpu.CompilerParams(needs_layout_passes=False),
        scratch_types=[
            pltpu.VMEM_SHARED((CHUNK + NDUMP, D), jnp.float32),
            pltpu.VMEM((EBLK,), jnp.int32),
            pltpu.VMEM((EBLK,), jnp.int32),
            pltpu.VMEM((KCAP_B,), jnp.int32),
            pltpu.VMEM((KCAP_B,), jnp.int32),
            pltpu.VMEM((B, D), jnp.float32),
            pltpu.VMEM((B, D), jnp.float32),
            pltpu.VMEM((CHUNK,), jnp.float32),
            pltpu.SemaphoreType.DMA,
            pltpu.SemaphoreType.DMA,
        ],
    )
    def k(xm_hbm, e0_hbm, e1_hbm, agg_hbm, deg_hbm,
          sh_chunk, e0s, e1s, kf, relf, row_buf, row_buf2, degloc,
          semA, semB):
        c = lax.axis_index("c")
        s = lax.axis_index("s")
        lane = lax.iota(jnp.int32, 16)
        zerof = jnp.zeros((16,), jnp.float32)
        safe_v = s * 16 + lane
        dump_v = jnp.full((16,), CHUNK, jnp.int32) + s

        def pass_body(p, _):
            lo = (c * NPASS + p) * CHUNK
            g0 = lo + s * RPW
            cg = jnp.minimum(g0, N - RPW)
            cl = cg - lo
            _fill2d(row_buf, B, zerof, lane)
            for t in range(RPW // B):
                pltpu.sync_copy(row_buf, sh_chunk.at[pl.ds(s * RPW + t * B, B)])
            rem = RPW % B
            if rem:
                pltpu.sync_copy(row_buf.at[pl.ds(0, rem)],
                                sh_chunk.at[pl.ds(s * RPW + (RPW // B) * B, rem)])

            def dz(i, _):
                degloc[pl.ds(i * 16, 16)] = zerof
                return 0

            lax.fori_loop(0, CHUNK // 16, dz, 0)
            plsc.subcore_barrier()

            def blk_body(blk, _):
                base = s * E_PW + blk * EBLK
                pltpu.sync_copy(e0_hbm.at[pl.ds(base, EBLK)], e0s)
                pltpu.sync_copy(e1_hbm.at[pl.ds(base, EBLK)], e1s)
                _prefill(kf, relf, KCAP_B // 16, safe_v, dump_v)

                def comp(i, cnt):
                    dv = e0s[pl.ds(i * 16, 16)]
                    relv = dv - lo
                    mask = (relv >= 0) & (relv < CHUNK)
                    mi = jnp.where(mask, 1, 0).astype(jnp.int32)
                    pc = plsc.cumsum(mi)
                    offs = cnt + pc - 1
                    sv = e1s[pl.ds(i * 16, 16)]
                    plsc.store_scatter(relf, [offs], relv, mask=mask)
                    plsc.store_scatter(kf, [offs], sv, mask=mask)
                    dcnt, lastm = plsc.scan_count(relv, mask=mask)
                    plsc.addupdate_scatter(
                        degloc, [relv], dcnt.astype(jnp.float32), mask=lastm)
                    return cnt + jnp.sum(mi)

                m = lax.fori_loop(0, EBLK // 16, comp, jnp.int32(0))

                nb = (m + B - 1) // B

                def gather_src(b):
                    return xm_hbm.at[kf.at[pl.ds(b * B, B)]]

                def drain(b, buf, sem, nxt_buf, nxt_sem):
                    pltpu.make_async_copy(gather_src(b), buf, sem).wait()

                    @pl.when(b + 1 < nb)
                    def _():
                        pltpu.async_copy(gather_src(b + 1), nxt_buf, nxt_sem)

                    pltpu.sync_copy(buf, sh_chunk.at[relf.at[pl.ds(b * B, B)]],
                                    add=True)

                @pl.when(nb > 0)
                def _():
                    pltpu.async_copy(gather_src(0), row_buf, semA)

                def bat(b, _):
                    @pl.when(b % 2 == 0)
                    def _():
                        drain(b, row_buf, semA, row_buf2, semB)

                    @pl.when(b % 2 == 1)
                    def _():
                        drain(b, row_buf2, semB, row_buf, semA)

                    return 0

                lax.fori_loop(0, nb, bat, 0)
                return 0

            lax.fori_loop(0, NBLK, blk_body, 0)
            plsc.subcore_barrier()
            _bounce(lambda t, n: sh_chunk.at[pl.ds(cl + t, n)],
                    lambda t, n: agg_hbm.at[pl.ds(cg + t, n)], row_buf, RPW)
            pltpu.sync_copy(degloc, deg_hbm.at[pl.ds(s * NPAD + lo, CHUNK)])
            plsc.subcore_barrier()
            return 0

        lax.fori_loop(0, NPASS, pass_body, 0)

    return k(x_mid, e0, e1)


def _proj_tc(ns_x, W1, b1):
    def body(ns_ref, w_ref, b_ref, o_ref):
        o_ref[...] = lax.dot_general(
            ns_ref[...], w_ref[...], (((1,), (1,)), ((), ())),
            preferred_element_type=jnp.float32) + b_ref[...]

    blk = 1000
    return pl.pallas_call(
        body,
        grid=(S // blk,),
        in_specs=[
            pl.BlockSpec((blk, D), lambda i: (i, 0)),
            pl.BlockSpec((D, D), lambda i: (0, 0)),
            pl.BlockSpec((1, D), lambda i: (0, 0)),
        ],
        out_specs=pl.BlockSpec((blk, D), lambda i: (i, 0)),
        out_shape=jax.ShapeDtypeStruct((S, D), jnp.float32),
    )(ns_x, W1, b1.reshape(1, D))


def _final_tc(x_mid, agg, deg16, W2, b2):
    def body(xm_ref, agg_ref, deg_ref, w_ref, b_ref, o_ref):
        acc = lax.dot_general(
            agg_ref[...], w_ref[...], (((1,), (1,)), ((), ())),
            preferred_element_type=jnp.float32)
        ones16 = jnp.ones((NSUB, 1), jnp.float32)
        degcol = lax.dot_general(
            deg_ref[...], ones16, (((0,), (0,)), ((), ())),
            preferred_element_type=jnp.float32)
        o_ref[...] = xm_ref[...] + acc + degcol * b_ref[...]

    blk = 2048
    return pl.pallas_call(
        body,
        grid=((N + blk - 1) // blk,),
        in_specs=[
            pl.BlockSpec((blk, D), lambda i: (i, 0)),
            pl.BlockSpec((blk, D), lambda i: (i, 0)),
            pl.BlockSpec((NSUB, blk), lambda i: (0, i)),
            pl.BlockSpec((D, D), lambda i: (0, 0)),
            pl.BlockSpec((1, D), lambda i: (0, 0)),
        ],
        out_specs=pl.BlockSpec((blk, D), lambda i: (i, 0)),
        out_shape=jax.ShapeDtypeStruct((N, D), jnp.float32),
    )(x_mid, agg, deg16, W2, b2.reshape(1, D))


def _sweep_a_carcass(x):
    mesh = plsc.VectorSubcoreMesh(core_axis_name="c", subcore_axis_name="s")

    @functools.partial(
        pl.kernel,
        out_type=jax.ShapeDtypeStruct((N, D), jnp.float32),
        mesh=mesh,
        compiler_params=pltpu.CompilerParams(needs_layout_passes=False),
        scratch_types=[
            pltpu.VMEM_SHARED((CHUNK + NDUMP, D), jnp.float32),
            pltpu.VMEM((B, D), jnp.float32),
        ],
    )
    def k(x_hbm, out_hbm, sh_chunk, row_buf):
        c = lax.axis_index("c")
        s = lax.axis_index("s")

        def pass_body(p, _):
            lo = (c * NPASS + p) * CHUNK
            g0 = lo + s * RPW
            cg = jnp.minimum(g0, N - RPW)
            cl = cg - lo
            _bounce(lambda t, n: x_hbm.at[pl.ds(cg + t, n)],
                    lambda t, n: sh_chunk.at[pl.ds(cl + t, n)], row_buf, RPW)
            plsc.subcore_barrier()
            _bounce(lambda t, n: sh_chunk.at[pl.ds(cl + t, n)],
                    lambda t, n: out_hbm.at[pl.ds(cg + t, n)], row_buf, RPW)
            plsc.subcore_barrier()
            return 0

        lax.fori_loop(0, NPASS, pass_body, 0)

    return k(x)


def kernel(x, new_supernode_x, supernode_edge_index, supernode_idx,
           graph_batch, W1, b1, W2, b2):
    del graph_batch
    sidx = supernode_idx.astype(jnp.int32)
    e0 = supernode_edge_index[0].astype(jnp.int32)
    e1 = supernode_edge_index[1].astype(jnp.int32)

    sidx_pad = jnp.concatenate(
        [sidx, jnp.full((SN_PAD - S,), _SENTINEL, jnp.int32)])
    proj1 = _proj_tc(new_supernode_x, W1, b1)
    x_mid = _sweep_a(x, proj1, sidx_pad)
    agg, deg_flat = _sweep_b(x_mid, e0, e1)
    deg16 = deg_flat.reshape(NSUB, NPAD)
    return _final_tc(x_mid, agg, deg16, W2, b2)


# speedup vs baseline: 2.9641x; 1.0145x over previous
import functools

import jax
import jax.numpy as jnp
from jax import lax
from jax.experimental import pallas as pl
from jax.experimental.pallas import tpu as pltpu
from jax.experimental.pallas import tpu_sc as plsc

N = 100000
S = 10000
E = 320000
D = 128

NSUB = 16
CHUNK = 10240
NPASS = 5
RPW = CHUNK // NSUB
NDUMP = 16

SN_PAD = 10240
SN_PW = SN_PAD // NSUB
E_PW = E // NSUB
EBLK = 2000
NBLK = E_PW // EBLK

B = 96
KCAP_A = ((SN_PW + B - 1) // B) * B
KCAP_B = ((EBLK + B - 1) // B) * B
NPAD = 2 * NPASS * CHUNK

_SENTINEL = 2 ** 30


def _fill2d(buf, nrows, val_v, lane):
    shift = (buf.shape[1] - 1).bit_length()
    assert buf.shape[1] == 1 << shift

    def body(i, _):
        f = i * 16 + lane
        plsc.store_scatter(buf, [f >> shift, f & (buf.shape[1] - 1)], val_v)
        return 0

    lax.fori_loop(0, nrows * buf.shape[1] // 16, body, 0)


def _bounce(src_at, dst_at, buf, nrows):
    nb = buf.shape[0]
    for t in range(0, nrows - nrows % nb, nb):
        pltpu.sync_copy(src_at(t, nb), buf)
        pltpu.sync_copy(buf, dst_at(t, nb))
    rem = nrows % nb
    if rem:
        t = nrows - rem
        pltpu.sync_copy(src_at(t, rem), buf.at[pl.ds(0, rem)])
        pltpu.sync_copy(buf.at[pl.ds(0, rem)], dst_at(t, rem))


def _bounce_pipe(src_at, dst_at, bufs, sems, nrows):
    nb = bufs[0].shape[0]
    hops = [(t, min(nb, nrows - t)) for t in range(0, nrows, nb)]
    for i, (t, n) in enumerate(hops):
        buf, sem = bufs[i % 2], sems[i % 2]
        pltpu.sync_copy(src_at(t, n), buf.at[pl.ds(0, n)])
        if i >= 1:
            tp, np_ = hops[i - 1]
            pltpu.make_async_copy(
                bufs[(i - 1) % 2].at[pl.ds(0, np_)], dst_at(tp, np_),
                sems[(i - 1) % 2]).wait()
        pltpu.async_copy(buf.at[pl.ds(0, n)], dst_at(t, n), sem)
    t, n = hops[-1]
    pltpu.make_async_copy(
        bufs[(len(hops) - 1) % 2].at[pl.ds(0, n)], dst_at(t, n),
        sems[(len(hops) - 1) % 2]).wait()


def _prefill(kf, relf, nvec, safe_v, dump_v):
    def body(i, _):
        kf[pl.ds(i * 16, 16)] = safe_v
        relf[pl.ds(i * 16, 16)] = dump_v
        return 0

    lax.fori_loop(0, nvec, body, 0)


def _compact(src_ids, dst_ids, kf, relf, nvec, lo, chunk):
    def body(i, cnt):
        dv = dst_ids(i)
        relv = dv - lo
        mask = (relv >= 0) & (relv < chunk)
        mi = jnp.where(mask, 1, 0).astype(jnp.int32)
        pc = plsc.cumsum(mi)
        offs = cnt + pc - 1
        sv = src_ids(i)
        plsc.store_scatter(relf, [offs], relv, mask=mask)
        plsc.store_scatter(kf, [offs], sv, mask=mask)
        return cnt + jnp.sum(mi)

    return lax.fori_loop(0, nvec, body, jnp.int32(0))


def _sweep_a(x, proj1, sidx_pad):
    mesh = plsc.VectorSubcoreMesh(core_axis_name="c", subcore_axis_name="s")

    @functools.partial(
        pl.kernel,
        out_type=jax.ShapeDtypeStruct((N, D), jnp.float32),
        mesh=mesh,
        compiler_params=pltpu.CompilerParams(needs_layout_passes=False),
        scratch_types=[
            pltpu.VMEM_SHARED((CHUNK + NDUMP, D), jnp.float32),
            pltpu.VMEM((SN_PW,), jnp.int32),
            pltpu.VMEM((KCAP_A,), jnp.int32),
            pltpu.VMEM((KCAP_A,), jnp.int32),
            pltpu.VMEM((256, D), jnp.float32),
            pltpu.SemaphoreType.DMA,
            pltpu.SemaphoreType.DMA,
        ],
    )
    def k(x_hbm, p1_hbm, sidx_hbm, out_hbm, sh_chunk, sids, kf, relf, row_buf,
          semA, semB):
        c = lax.axis_index("c")
        s = lax.axis_index("s")
        lane = lax.iota(jnp.int32, 16)
        pltpu.sync_copy(sidx_hbm.at[pl.ds(s * SN_PW, SN_PW)], sids)
        safe_v = s * 16 + lane
        dump_v = jnp.full((16,), CHUNK, jnp.int32) + s

        def pass_body(p, _):
            lo = (c * NPASS + p) * CHUNK
            g0 = lo + s * RPW
            cg = jnp.minimum(g0, N - RPW)
            cl = cg - lo
            _bounce(lambda t, n: x_hbm.at[pl.ds(cg + t, n)],
                    lambda t, n: sh_chunk.at[pl.ds(cl + t, n)], row_buf, RPW)
            _prefill(kf, relf, KCAP_A // 16, safe_v, dump_v)
            plsc.subcore_barrier()

            m = _compact(
                lambda i: s * SN_PW + i * 16 + lane,
                lambda i: sids[pl.ds(i * 16, 16)],
                kf, relf, SN_PW // 16, lo, CHUNK)

            nb = (m + B - 1) // B
            rbA = row_buf.at[pl.ds(0, B)]
            rbB = row_buf.at[pl.ds(B, B)]

            def gather_src(b):
                return p1_hbm.at[kf.at[pl.ds(b * B, B)]]

            def drain(b, buf, sem, nxt_buf, nxt_sem):
                pltpu.make_async_copy(gather_src(b), buf, sem).wait()

                @pl.when(b + 1 < nb)
                def _():
                    pltpu.async_copy(gather_src(b + 1), nxt_buf, nxt_sem)

                pltpu.sync_copy(buf, sh_chunk.at[relf.at[pl.ds(b * B, B)]],
                                add=True)

            @pl.when(nb > 0)
            def _():
                pltpu.async_copy(gather_src(0), rbA, semA)

            def bat(b, _):
                @pl.when(b % 2 == 0)
                def _():
                    drain(b, rbA, semA, rbB, semB)

                @pl.when(b % 2 == 1)
                def _():
                    drain(b, rbB, semB, rbA, semA)

                return 0

            lax.fori_loop(0, nb, bat, 0)
            plsc.subcore_barrier()
            _bounce(lambda t, n: sh_chunk.at[pl.ds(cl + t, n)],
                    lambda t, n: out_hbm.at[pl.ds(cg + t, n)], row_buf, RPW)
            plsc.subcore_barrier()
            return 0

        lax.fori_loop(0, NPASS, pass_body, 0)

    return k(x, proj1, sidx_pad)


def _sweep_b(x_mid, e0, e1):
    mesh = plsc.VectorSubcoreMesh(core_axis_name="c", subcore_axis_name="s")

    @functools.partial(
        pl.kernel,
        out_type=(
            jax.ShapeDtypeStruct((N, D), jnp.float32),
            jax.ShapeDtypeStruct((NSUB * NPAD,), jnp.float32),
        ),
        mesh=mesh,
        compiler_params=pltpu.CompilerParams(needs_layout_passes=False),
        scratch_types=[
            pltpu.VMEM_SHARED((CHUNK + NDUMP, D), jnp.float32),
            pltpu.VMEM((EBLK,), jnp.int32),
            pltpu.VMEM((EBLK,), jnp.int32),
            pltpu.VMEM((KCAP_B,), jnp.int32),
            pltpu.VMEM((KCAP_B,), jnp.int32),
            pltpu.VMEM((B, D), jnp.float32),
            pltpu.VMEM((B, D), jnp.float32),
            pltpu.VMEM((CHUNK,), jnp.float32),
            pltpu.SemaphoreType.DMA,
            pltpu.SemaphoreType.DMA,
        ],
    )
    def k(xm_hbm, e0_hbm, e1_hbm, agg_hbm, deg_hbm,
          sh_chunk, e0s, e1s, kf, relf, row_buf, row_buf2, degloc,
          semA, semB):
        c = lax.axis_index("c")
        s = lax.axis_index("s")
        lane = lax.iota(jnp.int32, 16)
        zerof = jnp.zeros((16,), jnp.float32)
        safe_v = s * 16 + lane
        dump_v = jnp.full((16,), CHUNK, jnp.int32) + s

        def pass_body(p, _):
            lo = (c * NPASS + p) * CHUNK
            g0 = lo + s * RPW
            cg = jnp.minimum(g0, N - RPW)
            cl = cg - lo
            _fill2d(row_buf, B, zerof, lane)
            for t in range(RPW // B):
                pltpu.sync_copy(row_buf, sh_chunk.at[pl.ds(s * RPW + t * B, B)])
            rem = RPW % B
            if rem:
                pltpu.sync_copy(row_buf.at[pl.ds(0, rem)],
                                sh_chunk.at[pl.ds(s * RPW + (RPW // B) * B, rem)])

            def dz(i, _):
                degloc[pl.ds(i * 16, 16)] = zerof
                return 0

            lax.fori_loop(0, CHUNK // 16, dz, 0)
            plsc.subcore_barrier()

            def blk_body(blk, _):
                base = s * E_PW + blk * EBLK
                pltpu.sync_copy(e0_hbm.at[pl.ds(base, EBLK)], e0s)
                pltpu.sync_copy(e1_hbm.at[pl.ds(base, EBLK)], e1s)
                _prefill(kf, relf, KCAP_B // 16, safe_v, dump_v)

                def comp(i, cnt):
                    dv = e0s[pl.ds(i * 16, 16)]
                    relv = dv - lo
                    mask = (relv >= 0) & (relv < CHUNK)
                    mi = jnp.where(mask, 1, 0).astype(jnp.int32)
                    pc = plsc.cumsum(mi)
                    offs = cnt + pc - 1
                    sv = e1s[pl.ds(i * 16, 16)]
                    plsc.store_scatter(relf, [offs], relv, mask=mask)
                    plsc.store_scatter(kf, [offs], sv, mask=mask)
                    dcnt, lastm = plsc.scan_count(relv, mask=mask)
                    plsc.addupdate_scatter(
                        degloc, [relv], dcnt.astype(jnp.float32), mask=lastm)
                    return cnt + jnp.sum(mi)

                m = lax.fori_loop(0, EBLK // 16, comp, jnp.int32(0))

                nb = (m + B - 1) // B

                def gather_src(b):
                    return xm_hbm.at[kf.at[pl.ds(b * B, B)]]

                def drain(b, buf, sem, nxt_buf, nxt_sem):
                    pltpu.make_async_copy(gather_src(b), buf, sem).wait()

                    @pl.when(b + 1 < nb)
                    def _():
                        pltpu.async_copy(gather_src(b + 1), nxt_buf, nxt_sem)

                    pltpu.sync_copy(buf, sh_chunk.at[relf.at[pl.ds(b * B, B)]],
                                    add=True)

                @pl.when(nb > 0)
                def _():
                    pltpu.async_copy(gather_src(0), row_buf, semA)

                def bat(b, _):
                    @pl.when(b % 2 == 0)
                    def _():
                        drain(b, row_buf, semA, row_buf2, semB)

                    @pl.when(b % 2 == 1)
                    def _():
                        drain(b, row_buf2, semB, row_buf, semA)

                    return 0

                lax.fori_loop(0, nb, bat, 0)
                return 0

            lax.fori_loop(0, NBLK, blk_body, 0)
            plsc.subcore_barrier()
            _bounce_pipe(lambda t, n: sh_chunk.at[pl.ds(cl + t, n)],
                         lambda t, n: agg_hbm.at[pl.ds(cg + t, n)],
                         (row_buf, row_buf2), (semA, semB), RPW)
            pltpu.sync_copy(degloc, deg_hbm.at[pl.ds(s * NPAD + lo, CHUNK)])
            plsc.subcore_barrier()
            return 0

        lax.fori_loop(0, NPASS, pass_body, 0)

    return k(x_mid, e0, e1)


def _proj_tc(ns_x, W1, b1):
    def body(ns_ref, w_ref, b_ref, o_ref):
        o_ref[...] = lax.dot_general(
            ns_ref[...], w_ref[...], (((1,), (1,)), ((), ())),
            preferred_element_type=jnp.float32) + b_ref[...]

    blk = 1000
    return pl.pallas_call(
        body,
        grid=(S // blk,),
        in_specs=[
            pl.BlockSpec((blk, D), lambda i: (i, 0)),
            pl.BlockSpec((D, D), lambda i: (0, 0)),
            pl.BlockSpec((1, D), lambda i: (0, 0)),
        ],
        out_specs=pl.BlockSpec((blk, D), lambda i: (i, 0)),
        out_shape=jax.ShapeDtypeStruct((S, D), jnp.float32),
    )(ns_x, W1, b1.reshape(1, D))


def _final_tc(x_mid, agg, deg16, W2, b2):
    def body(xm_ref, agg_ref, deg_ref, w_ref, b_ref, o_ref):
        acc = lax.dot_general(
            agg_ref[...], w_ref[...], (((1,), (1,)), ((), ())),
            preferred_element_type=jnp.float32)
        ones16 = jnp.ones((NSUB, 1), jnp.float32)
        degcol = lax.dot_general(
            deg_ref[...], ones16, (((0,), (0,)), ((), ())),
            preferred_element_type=jnp.float32)
        o_ref[...] = xm_ref[...] + acc + degcol * b_ref[...]

    blk = 2048
    return pl.pallas_call(
        body,
        grid=((N + blk - 1) // blk,),
        in_specs=[
            pl.BlockSpec((blk, D), lambda i: (i, 0)),
            pl.BlockSpec((blk, D), lambda i: (i, 0)),
            pl.BlockSpec((NSUB, blk), lambda i: (0, i)),
            pl.BlockSpec((D, D), lambda i: (0, 0)),
            pl.BlockSpec((1, D), lambda i: (0, 0)),
        ],
        out_specs=pl.BlockSpec((blk, D), lambda i: (i, 0)),
        out_shape=jax.ShapeDtypeStruct((N, D), jnp.float32),
    )(x_mid, agg, deg16, W2, b2.reshape(1, D))


def _sweep_a_carcass(x):
    mesh = plsc.VectorSubcoreMesh(core_axis_name="c", subcore_axis_name="s")

    @functools.partial(
        pl.kernel,
        out_type=jax.ShapeDtypeStruct((N, D), jnp.float32),
        mesh=mesh,
        compiler_params=pltpu.CompilerParams(needs_layout_passes=False),
        scratch_types=[
            pltpu.VMEM_SHARED((CHUNK + NDUMP, D), jnp.float32),
            pltpu.VMEM((B, D), jnp.float32),
        ],
    )
    def k(x_hbm, out_hbm, sh_chunk, row_buf):
        c = lax.axis_index("c")
        s = lax.axis_index("s")

        def pass_body(p, _):
            lo = (c * NPASS + p) * CHUNK
            g0 = lo + s * RPW
            cg = jnp.minimum(g0, N - RPW)
            cl = cg - lo
            _bounce(lambda t, n: x_hbm.at[pl.ds(cg + t, n)],
                    lambda t, n: sh_chunk.at[pl.ds(cl + t, n)], row_buf, RPW)
            plsc.subcore_barrier()
            _bounce(lambda t, n: sh_chunk.at[pl.ds(cl + t, n)],
                    lambda t, n: out_hbm.at[pl.ds(cg + t, n)], row_buf, RPW)
            plsc.subcore_barrier()
            return 0

        lax.fori_loop(0, NPASS, pass_body, 0)

    return k(x)


def kernel(x, new_supernode_x, supernode_edge_index, supernode_idx,
           graph_batch, W1, b1, W2, b2):
    del graph_batch
    sidx = supernode_idx.astype(jnp.int32)
    e0 = supernode_edge_index[0].astype(jnp.int32)
    e1 = supernode_edge_index[1].astype(jnp.int32)

    sidx_pad = jnp.concatenate(
        [sidx, jnp.full((SN_PAD - S,), _SENTINEL, jnp.int32)])
    proj1 = _proj_tc(new_supernode_x, W1, b1)
    x_mid = _sweep_a(x, proj1, sidx_pad)
    agg, deg_flat = _sweep_b(x_mid, e0, e1)
    deg16 = deg_flat.reshape(NSUB, NPAD)
    return _final_tc(x_mid, agg, deg16, W2, b2)


# speedup vs baseline: 3.0724x; 1.0365x over previous
import functools

import jax
import jax.numpy as jnp
from jax import lax
from jax.experimental import pallas as pl
from jax.experimental.pallas import tpu as pltpu
from jax.experimental.pallas import tpu_sc as plsc

N = 100000
S = 10000
E = 320000
D = 128

NSUB = 16
CHUNK = 10240
NPASS = 5
RPW = CHUNK // NSUB
NDUMP = 16

SN_PAD = 10240
SN_PW = SN_PAD // NSUB
E_PW = E // NSUB
EBLK = 2000
NBLK = E_PW // EBLK

B = 96
KCAP_A = ((SN_PW + B - 1) // B) * B
KCAP_B = ((EBLK + B - 1) // B) * B
NPAD = 2 * NPASS * CHUNK

_SENTINEL = 2 ** 30


def _fill2d(buf, nrows, val_v, lane):
    shift = (buf.shape[1] - 1).bit_length()
    assert buf.shape[1] == 1 << shift

    def body(i, _):
        f = i * 16 + lane
        plsc.store_scatter(buf, [f >> shift, f & (buf.shape[1] - 1)], val_v)
        return 0

    lax.fori_loop(0, nrows * buf.shape[1] // 16, body, 0)


def _bounce(src_at, dst_at, buf, nrows):
    nb = buf.shape[0]
    for t in range(0, nrows - nrows % nb, nb):
        pltpu.sync_copy(src_at(t, nb), buf)
        pltpu.sync_copy(buf, dst_at(t, nb))
    rem = nrows % nb
    if rem:
        t = nrows - rem
        pltpu.sync_copy(src_at(t, rem), buf.at[pl.ds(0, rem)])
        pltpu.sync_copy(buf.at[pl.ds(0, rem)], dst_at(t, rem))


def _bounce_pipe(src_at, dst_at, bufs, sems, nrows):
    nb = bufs[0].shape[0]
    hops = [(t, min(nb, nrows - t)) for t in range(0, nrows, nb)]
    for i, (t, n) in enumerate(hops):
        buf, sem = bufs[i % 2], sems[i % 2]
        pltpu.sync_copy(src_at(t, n), buf.at[pl.ds(0, n)])
        if i >= 1:
            tp, np_ = hops[i - 1]
            pltpu.make_async_copy(
                bufs[(i - 1) % 2].at[pl.ds(0, np_)], dst_at(tp, np_),
                sems[(i - 1) % 2]).wait()
        pltpu.async_copy(buf.at[pl.ds(0, n)], dst_at(t, n), sem)
    t, n = hops[-1]
    pltpu.make_async_copy(
        bufs[(len(hops) - 1) % 2].at[pl.ds(0, n)], dst_at(t, n),
        sems[(len(hops) - 1) % 2]).wait()


def _pad_tail(kf, relf, m, cap, safe_v, dump_v, lane):
    for j in range(8):
        idx = m + j * 16 + lane
        msk = idx < cap
        plsc.store_scatter(kf, [idx], safe_v, mask=msk)
        plsc.store_scatter(relf, [idx], dump_v, mask=msk)


def _compact(src_ids, dst_ids, kf, relf, nvec, lo, chunk):
    def body(i, cnt):
        dv = dst_ids(i)
        relv = dv - lo
        mask = (relv >= 0) & (relv < chunk)
        mi = jnp.where(mask, 1, 0).astype(jnp.int32)
        pc = plsc.cumsum(mi)
        offs = cnt + pc - 1
        sv = src_ids(i)
        plsc.store_scatter(relf, [offs], relv, mask=mask)
        plsc.store_scatter(kf, [offs], sv, mask=mask)
        return cnt + jnp.sum(mi)

    return lax.fori_loop(0, nvec, body, jnp.int32(0))


def _sweep_a(x, proj1, sidx_pad):
    mesh = plsc.VectorSubcoreMesh(core_axis_name="c", subcore_axis_name="s")

    @functools.partial(
        pl.kernel,
        out_type=jax.ShapeDtypeStruct((N, D), jnp.float32),
        mesh=mesh,
        compiler_params=pltpu.CompilerParams(needs_layout_passes=False),
        scratch_types=[
            pltpu.VMEM_SHARED((CHUNK + NDUMP, D), jnp.float32),
            pltpu.VMEM((SN_PW,), jnp.int32),
            pltpu.VMEM((KCAP_A,), jnp.int32),
            pltpu.VMEM((KCAP_A,), jnp.int32),
            pltpu.VMEM((256, D), jnp.float32),
            pltpu.SemaphoreType.DMA,
            pltpu.SemaphoreType.DMA,
        ],
    )
    def k(x_hbm, p1_hbm, sidx_hbm, out_hbm, sh_chunk, sids, kf, relf, row_buf,
          semA, semB):
        c = lax.axis_index("c")
        s = lax.axis_index("s")
        lane = lax.iota(jnp.int32, 16)
        pltpu.sync_copy(sidx_hbm.at[pl.ds(s * SN_PW, SN_PW)], sids)
        safe_v = s * 16 + lane
        dump_v = jnp.full((16,), CHUNK, jnp.int32) + s

        def pass_body(p, _):
            lo = (c * NPASS + p) * CHUNK
            g0 = lo + s * RPW
            cg = jnp.minimum(g0, N - RPW)
            cl = cg - lo
            _bounce(lambda t, n: x_hbm.at[pl.ds(cg + t, n)],
                    lambda t, n: sh_chunk.at[pl.ds(cl + t, n)], row_buf, RPW)
            plsc.subcore_barrier()

            m = _compact(
                lambda i: s * SN_PW + i * 16 + lane,
                lambda i: sids[pl.ds(i * 16, 16)],
                kf, relf, SN_PW // 16, lo, CHUNK)
            _pad_tail(kf, relf, m, KCAP_A, safe_v, dump_v, lane)

            nb = (m + B - 1) // B
            rbA = row_buf.at[pl.ds(0, B)]
            rbB = row_buf.at[pl.ds(B, B)]

            def gather_src(b):
                return p1_hbm.at[kf.at[pl.ds(b * B, B)]]

            def drain(b, buf, sem, nxt_buf, nxt_sem):
                pltpu.make_async_copy(gather_src(b), buf, sem).wait()

                @pl.when(b + 1 < nb)
                def _():
                    pltpu.async_copy(gather_src(b + 1), nxt_buf, nxt_sem)

                pltpu.sync_copy(buf, sh_chunk.at[relf.at[pl.ds(b * B, B)]],
                                add=True)

            @pl.when(nb > 0)
            def _():
                pltpu.async_copy(gather_src(0), rbA, semA)

            def bat(b, _):
                @pl.when(b % 2 == 0)
                def _():
                    drain(b, rbA, semA, rbB, semB)

                @pl.when(b % 2 == 1)
                def _():
                    drain(b, rbB, semB, rbA, semA)

                return 0

            lax.fori_loop(0, nb, bat, 0)
            plsc.subcore_barrier()
            _bounce(lambda t, n: sh_chunk.at[pl.ds(cl + t, n)],
                    lambda t, n: out_hbm.at[pl.ds(cg + t, n)], row_buf, RPW)
            plsc.subcore_barrier()
            return 0

        lax.fori_loop(0, NPASS, pass_body, 0)

    return k(x, proj1, sidx_pad)


def _sweep_b(x_mid, e0, e1):
    mesh = plsc.VectorSubcoreMesh(core_axis_name="c", subcore_axis_name="s")

    @functools.partial(
        pl.kernel,
        out_type=(
            jax.ShapeDtypeStruct((N, D), jnp.float32),
            jax.ShapeDtypeStruct((NSUB * NPAD,), jnp.float32),
        ),
        mesh=mesh,
        compiler_params=pltpu.CompilerParams(needs_layout_passes=False),
        scratch_types=[
            pltpu.VMEM_SHARED((CHUNK + NDUMP, D), jnp.float32),
            pltpu.VMEM((EBLK,), jnp.int32),
            pltpu.VMEM((EBLK,), jnp.int32),
            pltpu.VMEM((KCAP_B,), jnp.int32),
            pltpu.VMEM((KCAP_B,), jnp.int32),
            pltpu.VMEM((B, D), jnp.float32),
            pltpu.VMEM((B, D), jnp.float32),
            pltpu.VMEM((CHUNK,), jnp.float32),
            pltpu.SemaphoreType.DMA,
            pltpu.SemaphoreType.DMA,
        ],
    )
    def k(xm_hbm, e0_hbm, e1_hbm, agg_hbm, deg_hbm,
          sh_chunk, e0s, e1s, kf, relf, row_buf, row_buf2, degloc,
          semA, semB):
        c = lax.axis_index("c")
        s = lax.axis_index("s")
        lane = lax.iota(jnp.int32, 16)
        zerof = jnp.zeros((16,), jnp.float32)
        safe_v = s * 16 + lane
        dump_v = jnp.full((16,), CHUNK, jnp.int32) + s

        def pass_body(p, _):
            lo = (c * NPASS + p) * CHUNK
            g0 = lo + s * RPW
            cg = jnp.minimum(g0, N - RPW)
            cl = cg - lo
            _fill2d(row_buf, B, zerof, lane)
            for t in range(RPW // B):
                pltpu.sync_copy(row_buf, sh_chunk.at[pl.ds(s * RPW + t * B, B)])
            rem = RPW % B
            if rem:
                pltpu.sync_copy(row_buf.at[pl.ds(0, rem)],
                                sh_chunk.at[pl.ds(s * RPW + (RPW // B) * B, rem)])

            def dz(i, _):
                degloc[pl.ds(i * 16, 16)] = zerof
                return 0

            lax.fori_loop(0, CHUNK // 16, dz, 0)
            plsc.subcore_barrier()

            def blk_body(blk, _):
                base = s * E_PW + blk * EBLK
                pltpu.sync_copy(e0_hbm.at[pl.ds(base, EBLK)], e0s)
                pltpu.sync_copy(e1_hbm.at[pl.ds(base, EBLK)], e1s)

                def comp(i, cnt):
                    dv = e0s[pl.ds(i * 16, 16)]
                    relv = dv - lo
                    mask = (relv >= 0) & (relv < CHUNK)
                    mi = jnp.where(mask, 1, 0).astype(jnp.int32)
                    pc = plsc.cumsum(mi)
                    offs = cnt + pc - 1
                    sv = e1s[pl.ds(i * 16, 16)]
                    plsc.store_scatter(relf, [offs], relv, mask=mask)
                    plsc.store_scatter(kf, [offs], sv, mask=mask)
                    dcnt, lastm = plsc.scan_count(relv, mask=mask)
                    plsc.addupdate_scatter(
                        degloc, [relv], dcnt.astype(jnp.float32), mask=lastm)
                    return cnt + jnp.sum(mi)

                m = lax.fori_loop(0, EBLK // 16, comp, jnp.int32(0))
                _pad_tail(kf, relf, m, KCAP_B, safe_v, dump_v, lane)

                nb = (m + B - 1) // B

                def gather_src(b):
                    return xm_hbm.at[kf.at[pl.ds(b * B, B)]]

                def drain(b, buf, sem, nxt_buf, nxt_sem):
                    pltpu.make_async_copy(gather_src(b), buf, sem).wait()

                    @pl.when(b + 1 < nb)
                    def _():
                        pltpu.async_copy(gather_src(b + 1), nxt_buf, nxt_sem)

                    pltpu.sync_copy(buf, sh_chunk.at[relf.at[pl.ds(b * B, B)]],
                                    add=True)

                @pl.when(nb > 0)
                def _():
                    pltpu.async_copy(gather_src(0), row_buf, semA)

                def bat(b, _):
                    @pl.when(b % 2 == 0)
                    def _():
                        drain(b, row_buf, semA, row_buf2, semB)

                    @pl.when(b % 2 == 1)
                    def _():
                        drain(b, row_buf2, semB, row_buf, semA)

                    return 0

                lax.fori_loop(0, nb, bat, 0)
                return 0

            lax.fori_loop(0, NBLK, blk_body, 0)
            plsc.subcore_barrier()
            _bounce_pipe(lambda t, n: sh_chunk.at[pl.ds(cl + t, n)],
                         lambda t, n: agg_hbm.at[pl.ds(cg + t, n)],
                         (row_buf, row_buf2), (semA, semB), RPW)
            pltpu.sync_copy(degloc, deg_hbm.at[pl.ds(s * NPAD + lo, CHUNK)])
            plsc.subcore_barrier()
            return 0

        lax.fori_loop(0, NPASS, pass_body, 0)

    return k(x_mid, e0, e1)


def _proj_tc(ns_x, W1, b1):
    def body(ns_ref, w_ref, b_ref, o_ref):
        o_ref[...] = lax.dot_general(
            ns_ref[...], w_ref[...], (((1,), (1,)), ((), ())),
            preferred_element_type=jnp.float32) + b_ref[...]

    blk = 1000
    return pl.pallas_call(
        body,
        grid=(S // blk,),
        in_specs=[
            pl.BlockSpec((blk, D), lambda i: (i, 0)),
            pl.BlockSpec((D, D), lambda i: (0, 0)),
            pl.BlockSpec((1, D), lambda i: (0, 0)),
        ],
        out_specs=pl.BlockSpec((blk, D), lambda i: (i, 0)),
        out_shape=jax.ShapeDtypeStruct((S, D), jnp.float32),
    )(ns_x, W1, b1.reshape(1, D))


def _final_tc(x_mid, agg, deg16, W2, b2):
    def body(xm_ref, agg_ref, deg_ref, w_ref, b_ref, o_ref):
        acc = lax.dot_general(
            agg_ref[...], w_ref[...], (((1,), (1,)), ((), ())),
            preferred_element_type=jnp.float32)
        ones16 = jnp.ones((NSUB, 1), jnp.float32)
        degcol = lax.dot_general(
            deg_ref[...], ones16, (((0,), (0,)), ((), ())),
            preferred_element_type=jnp.float32)
        o_ref[...] = xm_ref[...] + acc + degcol * b_ref[...]

    blk = 2048
    return pl.pallas_call(
        body,
        grid=((N + blk - 1) // blk,),
        in_specs=[
            pl.BlockSpec((blk, D), lambda i: (i, 0)),
            pl.BlockSpec((blk, D), lambda i: (i, 0)),
            pl.BlockSpec((NSUB, blk), lambda i: (0, i)),
            pl.BlockSpec((D, D), lambda i: (0, 0)),
            pl.BlockSpec((1, D), lambda i: (0, 0)),
        ],
        out_specs=pl.BlockSpec((blk, D), lambda i: (i, 0)),
        out_shape=jax.ShapeDtypeStruct((N, D), jnp.float32),
    )(x_mid, agg, deg16, W2, b2.reshape(1, D))


def _sweep_a_carcass(x):
    mesh = plsc.VectorSubcoreMesh(core_axis_name="c", subcore_axis_name="s")

    @functools.partial(
        pl.kernel,
        out_type=jax.ShapeDtypeStruct((N, D), jnp.float32),
        mesh=mesh,
        compiler_params=pltpu.CompilerParams(needs_layout_passes=False),
        scratch_types=[
            pltpu.VMEM_SHARED((CHUNK + NDUMP, D), jnp.float32),
            pltpu.VMEM((B, D), jnp.float32),
        ],
    )
    def k(x_hbm, out_hbm, sh_chunk, row_buf):
        c = lax.axis_index("c")
        s = lax.axis_index("s")

        def pass_body(p, _):
            lo = (c * NPASS + p) * CHUNK
            g0 = lo + s * RPW
            cg = jnp.minimum(g0, N - RPW)
            cl = cg - lo
            _bounce(lambda t, n: x_hbm.at[pl.ds(cg + t, n)],
                    lambda t, n: sh_chunk.at[pl.ds(cl + t, n)], row_buf, RPW)
            plsc.subcore_barrier()
            _bounce(lambda t, n: sh_chunk.at[pl.ds(cl + t, n)],
                    lambda t, n: out_hbm.at[pl.ds(cg + t, n)], row_buf, RPW)
            plsc.subcore_barrier()
            return 0

        lax.fori_loop(0, NPASS, pass_body, 0)

    return k(x)


def kernel(x, new_supernode_x, supernode_edge_index, supernode_idx,
           graph_batch, W1, b1, W2, b2):
    del graph_batch
    sidx = supernode_idx.astype(jnp.int32)
    e0 = supernode_edge_index[0].astype(jnp.int32)
    e1 = supernode_edge_index[1].astype(jnp.int32)

    sidx_pad = jnp.concatenate(
        [sidx, jnp.full((SN_PAD - S,), _SENTINEL, jnp.int32)])
    proj1 = _proj_tc(new_supernode_x, W1, b1)
    x_mid = _sweep_a(x, proj1, sidx_pad)
    agg, deg_flat = _sweep_b(x_mid, e0, e1)
    deg16 = deg_flat.reshape(NSUB, NPAD)
    return _final_tc(x_mid, agg, deg16, W2, b2)


# speedup vs baseline: 3.0734x; 1.0003x over previous
import functools

import jax
import jax.numpy as jnp
from jax import lax
from jax.experimental import pallas as pl
from jax.experimental.pallas import tpu as pltpu
from jax.experimental.pallas import tpu_sc as plsc

N = 100000
S = 10000
E = 320000
D = 128

NSUB = 16
CHUNK = 10240
NPASS = 5
RPW = CHUNK // NSUB
NDUMP = 16

SN_PAD = 10240
SN_PW = SN_PAD // NSUB
E_PW = E // NSUB
EBLK = 2000
NBLK = E_PW // EBLK

B = 96
KCAP_A = ((SN_PW + B - 1) // B) * B
KCAP_B = ((EBLK + B - 1) // B) * B
NPAD = 2 * NPASS * CHUNK

_SENTINEL = 2 ** 30


def _fill2d(buf, nrows, val_v, lane):
    shift = (buf.shape[1] - 1).bit_length()
    assert buf.shape[1] == 1 << shift

    def body(i, _):
        f = i * 16 + lane
        plsc.store_scatter(buf, [f >> shift, f & (buf.shape[1] - 1)], val_v)
        return 0

    lax.fori_loop(0, nrows * buf.shape[1] // 16, body, 0)


def _bounce(src_at, dst_at, buf, nrows):
    nb = buf.shape[0]
    for t in range(0, nrows - nrows % nb, nb):
        pltpu.sync_copy(src_at(t, nb), buf)
        pltpu.sync_copy(buf, dst_at(t, nb))
    rem = nrows % nb
    if rem:
        t = nrows - rem
        pltpu.sync_copy(src_at(t, rem), buf.at[pl.ds(0, rem)])
        pltpu.sync_copy(buf.at[pl.ds(0, rem)], dst_at(t, rem))


def _bounce_pipe(src_at, dst_at, bufs, sems, nrows):
    nb = bufs[0].shape[0]
    hops = [(t, min(nb, nrows - t)) for t in range(0, nrows, nb)]
    for i, (t, n) in enumerate(hops):
        buf, sem = bufs[i % 2], sems[i % 2]
        pltpu.sync_copy(src_at(t, n), buf.at[pl.ds(0, n)])
        if i >= 1:
            tp, np_ = hops[i - 1]
            pltpu.make_async_copy(
                bufs[(i - 1) % 2].at[pl.ds(0, np_)], dst_at(tp, np_),
                sems[(i - 1) % 2]).wait()
        pltpu.async_copy(buf.at[pl.ds(0, n)], dst_at(t, n), sem)
    t, n = hops[-1]
    pltpu.make_async_copy(
        bufs[(len(hops) - 1) % 2].at[pl.ds(0, n)], dst_at(t, n),
        sems[(len(hops) - 1) % 2]).wait()


def _pad_tail(kf, relf, m, cap, safe_v, dump_v, lane):
    for j in range(8):
        idx = m + j * 16 + lane
        msk = idx < cap
        plsc.store_scatter(kf, [idx], safe_v, mask=msk)
        plsc.store_scatter(relf, [idx], dump_v, mask=msk)


def _compact(src_ids, dst_ids, kf, relf, nvec, lo, chunk):
    def body(i, cnt):
        dv = dst_ids(i)
        relv = dv - lo
        mask = (relv >= 0) & (relv < chunk)
        mi = jnp.where(mask, 1, 0).astype(jnp.int32)
        pc = plsc.cumsum(mi)
        offs = cnt + pc - 1
        sv = src_ids(i)
        plsc.store_scatter(relf, [offs], relv, mask=mask)
        plsc.store_scatter(kf, [offs], sv, mask=mask)
        return cnt + jnp.sum(mi)

    return lax.fori_loop(0, nvec, body, jnp.int32(0))


def _sweep_a(x, proj1, sidx_pad):
    mesh = plsc.VectorSubcoreMesh(core_axis_name="c", subcore_axis_name="s")

    @functools.partial(
        pl.kernel,
        out_type=jax.ShapeDtypeStruct((N, D), jnp.float32),
        mesh=mesh,
        compiler_params=pltpu.CompilerParams(needs_layout_passes=False),
        scratch_types=[
            pltpu.VMEM_SHARED((CHUNK + NDUMP, D), jnp.float32),
            pltpu.VMEM((SN_PW,), jnp.int32),
            pltpu.VMEM((KCAP_A,), jnp.int32),
            pltpu.VMEM((KCAP_A,), jnp.int32),
            pltpu.VMEM((256, D), jnp.float32),
            pltpu.SemaphoreType.DMA,
            pltpu.SemaphoreType.DMA,
        ],
    )
    def k(x_hbm, p1_hbm, sidx_hbm, out_hbm, sh_chunk, sids, kf, relf, row_buf,
          semA, semB):
        c = lax.axis_index("c")
        s = lax.axis_index("s")
        lane = lax.iota(jnp.int32, 16)
        pltpu.sync_copy(sidx_hbm.at[pl.ds(s * SN_PW, SN_PW)], sids)
        safe_v = s * 16 + lane
        dump_v = jnp.full((16,), CHUNK, jnp.int32) + s

        def pass_body(p, _):
            lo = (c * NPASS + p) * CHUNK
            g0 = lo + s * RPW
            cg = jnp.minimum(g0, N - RPW)
            cl = cg - lo
            _bounce(lambda t, n: x_hbm.at[pl.ds(cg + t, n)],
                    lambda t, n: sh_chunk.at[pl.ds(cl + t, n)], row_buf, RPW)
            plsc.subcore_barrier()

            m = _compact(
                lambda i: s * SN_PW + i * 16 + lane,
                lambda i: sids[pl.ds(i * 16, 16)],
                kf, relf, SN_PW // 16, lo, CHUNK)
            _pad_tail(kf, relf, m, KCAP_A, safe_v, dump_v, lane)

            nb = (m + B - 1) // B
            rbA = row_buf.at[pl.ds(0, B)]
            rbB = row_buf.at[pl.ds(B, B)]

            def gather_src(b):
                return p1_hbm.at[kf.at[pl.ds(b * B, B)]]

            def drain(b, buf, sem, nxt_buf, nxt_sem):
                pltpu.make_async_copy(gather_src(b), buf, sem).wait()

                @pl.when(b + 1 < nb)
                def _():
                    pltpu.async_copy(gather_src(b + 1), nxt_buf, nxt_sem)

                pltpu.sync_copy(buf, sh_chunk.at[relf.at[pl.ds(b * B, B)]],
                                add=True)

            @pl.when(nb > 0)
            def _():
                pltpu.async_copy(gather_src(0), rbA, semA)

            def bat(b, _):
                @pl.when(b % 2 == 0)
                def _():
                    drain(b, rbA, semA, rbB, semB)

                @pl.when(b % 2 == 1)
                def _():
                    drain(b, rbB, semB, rbA, semA)

                return 0

            lax.fori_loop(0, nb, bat, 0)
            plsc.subcore_barrier()
            _bounce(lambda t, n: sh_chunk.at[pl.ds(cl + t, n)],
                    lambda t, n: out_hbm.at[pl.ds(cg + t, n)], row_buf, RPW)
            plsc.subcore_barrier()
            return 0

        lax.fori_loop(0, NPASS, pass_body, 0)

    return k(x, proj1, sidx_pad)


def _sweep_b(x_mid, e0, e1):
    mesh = plsc.VectorSubcoreMesh(core_axis_name="c", subcore_axis_name="s")

    @functools.partial(
        pl.kernel,
        out_type=(
            jax.ShapeDtypeStruct((N, D), jnp.float32),
            jax.ShapeDtypeStruct((NSUB * NPAD,), jnp.float32),
        ),
        mesh=mesh,
        compiler_params=pltpu.CompilerParams(needs_layout_passes=False),
        scratch_types=[
            pltpu.VMEM_SHARED((CHUNK + NDUMP, D), jnp.float32),
            pltpu.VMEM((EBLK,), jnp.int32),
            pltpu.VMEM((EBLK,), jnp.int32),
            pltpu.VMEM((KCAP_B,), jnp.int32),
            pltpu.VMEM((KCAP_B,), jnp.int32),
            pltpu.VMEM((B, D), jnp.float32),
            pltpu.VMEM((B, D), jnp.float32),
            pltpu.VMEM((CHUNK,), jnp.float32),
            pltpu.SemaphoreType.DMA,
            pltpu.SemaphoreType.DMA,
        ],
    )
    def k(xm_hbm, e0_hbm, e1_hbm, agg_hbm, deg_hbm,
          sh_chunk, e0s, e1s, kf, relf, row_buf, row_buf2, degloc,
          semA, semB):
        c = lax.axis_index("c")
        s = lax.axis_index("s")
        lane = lax.iota(jnp.int32, 16)
        zerof = jnp.zeros((16,), jnp.float32)
        safe_v = s * 16 + lane
        dump_v = jnp.full((16,), CHUNK, jnp.int32) + s

        def pass_body(p, _):
            lo = (c * NPASS + p) * CHUNK
            g0 = lo + s * RPW
            cg = jnp.minimum(g0, N - RPW)
            cl = cg - lo
            _fill2d(row_buf, B, zerof, lane)
            for t in range(RPW // B):
                pltpu.sync_copy(row_buf, sh_chunk.at[pl.ds(s * RPW + t * B, B)])
            rem = RPW % B
            if rem:
                pltpu.sync_copy(row_buf.at[pl.ds(0, rem)],
                                sh_chunk.at[pl.ds(s * RPW + (RPW // B) * B, rem)])

            def dz(i, _):
                degloc[pl.ds(i * 16, 16)] = zerof
                return 0

            lax.fori_loop(0, CHUNK // 16, dz, 0)
            plsc.subcore_barrier()

            def blk_body(blk, _):
                base = s * E_PW + blk * EBLK
                pltpu.sync_copy(e0_hbm.at[pl.ds(base, EBLK)], e0s)
                pltpu.sync_copy(e1_hbm.at[pl.ds(base, EBLK)], e1s)

                def comp(i, cnt):
                    dv = e0s[pl.ds(i * 16, 16)]
                    relv = dv - lo
                    mask = (relv >= 0) & (relv < CHUNK)
                    mi = jnp.where(mask, 1, 0).astype(jnp.int32)
                    pc = plsc.cumsum(mi)
                    offs = cnt + pc - 1
                    sv = e1s[pl.ds(i * 16, 16)]
                    plsc.store_scatter(relf, [offs], relv, mask=mask)
                    plsc.store_scatter(kf, [offs], sv, mask=mask)
                    dcnt, lastm = plsc.scan_count(relv, mask=mask)
                    plsc.addupdate_scatter(
                        degloc, [relv], dcnt.astype(jnp.float32), mask=lastm)
                    return cnt + jnp.sum(mi)

                m = lax.fori_loop(0, EBLK // 16, comp, jnp.int32(0))
                _pad_tail(kf, relf, m, KCAP_B, safe_v, dump_v, lane)

                nb = (m + B - 1) // B

                def gather_src(b):
                    return xm_hbm.at[kf.at[pl.ds(b * B, B)]]

                def drain(b, buf, sem, nxt_buf, nxt_sem):
                    pltpu.make_async_copy(gather_src(b), buf, sem).wait()

                    @pl.when(b + 1 < nb)
                    def _():
                        pltpu.async_copy(gather_src(b + 1), nxt_buf, nxt_sem)

                    pltpu.sync_copy(buf, sh_chunk.at[relf.at[pl.ds(b * B, B)]],
                                    add=True)

                @pl.when(nb > 0)
                def _():
                    pltpu.async_copy(gather_src(0), row_buf, semA)

                def bat(b, _):
                    @pl.when(b % 2 == 0)
                    def _():
                        drain(b, row_buf, semA, row_buf2, semB)

                    @pl.when(b % 2 == 1)
                    def _():
                        drain(b, row_buf2, semB, row_buf, semA)

                    return 0

                lax.fori_loop(0, nb, bat, 0)
                return 0

            lax.fori_loop(0, NBLK, blk_body, 0)
            plsc.subcore_barrier()
            _bounce_pipe(lambda t, n: sh_chunk.at[pl.ds(cl + t, n)],
                         lambda t, n: agg_hbm.at[pl.ds(cg + t, n)],
                         (row_buf, row_buf2), (semA, semB), RPW)
            pltpu.sync_copy(degloc, deg_hbm.at[pl.ds(s * NPAD + lo, CHUNK)])
            plsc.subcore_barrier()
            return 0

        lax.fori_loop(0, NPASS, pass_body, 0)

    return k(x_mid, e0, e1)


def _proj_tc(ns_x, W1, b1):
    def body(ns_ref, w_ref, b_ref, o_ref):
        o_ref[...] = lax.dot_general(
            ns_ref[...], w_ref[...], (((1,), (1,)), ((), ())),
            preferred_element_type=jnp.float32) + b_ref[...]

    blk = 1000
    return pl.pallas_call(
        body,
        grid=(S // blk,),
        in_specs=[
            pl.BlockSpec((blk, D), lambda i: (i, 0)),
            pl.BlockSpec((D, D), lambda i: (0, 0)),
            pl.BlockSpec((1, D), lambda i: (0, 0)),
        ],
        out_specs=pl.BlockSpec((blk, D), lambda i: (i, 0)),
        out_shape=jax.ShapeDtypeStruct((S, D), jnp.float32),
    )(ns_x, W1, b1.reshape(1, D))


def _final_tc(x_mid, agg, deg16, W2, b2):
    def body(xm_ref, agg_ref, deg_ref, w_ref, b_ref, o_ref):
        acc = lax.dot_general(
            agg_ref[...], w_ref[...], (((1,), (1,)), ((), ())),
            preferred_element_type=jnp.float32)
        ones16 = jnp.ones((NSUB, 1), jnp.float32)
        degcol = lax.dot_general(
            deg_ref[...], ones16, (((0,), (0,)), ((), ())),
            preferred_element_type=jnp.float32)
        o_ref[...] = xm_ref[...] + acc + degcol * b_ref[...]

    blk = 2048
    return pl.pallas_call(
        body,
        grid=((N + blk - 1) // blk,),
        in_specs=[
            pl.BlockSpec((blk, D), lambda i: (i, 0)),
            pl.BlockSpec((blk, D), lambda i: (i, 0)),
            pl.BlockSpec((NSUB, blk), lambda i: (0, i)),
            pl.BlockSpec((D, D), lambda i: (0, 0)),
            pl.BlockSpec((1, D), lambda i: (0, 0)),
        ],
        out_specs=pl.BlockSpec((blk, D), lambda i: (i, 0)),
        out_shape=jax.ShapeDtypeStruct((N, D), jnp.float32),
    )(x_mid, agg, deg16, W2, b2.reshape(1, D))


def kernel(x, new_supernode_x, supernode_edge_index, supernode_idx,
           graph_batch, W1, b1, W2, b2):
    del graph_batch
    sidx = supernode_idx.astype(jnp.int32)
    e0 = supernode_edge_index[0].astype(jnp.int32)
    e1 = supernode_edge_index[1].astype(jnp.int32)

    sidx_pad = jnp.concatenate(
        [sidx, jnp.full((SN_PAD - S,), _SENTINEL, jnp.int32)])
    proj1 = _proj_tc(new_supernode_x, W1, b1)
    x_mid = _sweep_a(x, proj1, sidx_pad)
    agg, deg_flat = _sweep_b(x_mid, e0, e1)
    deg16 = deg_flat.reshape(NSUB, NPAD)
    return _final_tc(x_mid, agg, deg16, W2, b2)
